# baseline (speedup 1.0000x reference)
import jax
import jax.numpy as jnp
from jax.experimental import pallas as pl
from jax.experimental.pallas import tpu as pltpu
from jax.experimental.pallas import tpu_sc as plsc

BATCH = 16384
NF = 64


_NC = 2
_NS = 16
_NW = _NC * _NS


def _sc_gather_pair(users, items, user_table, item_table):
    mesh = plsc.VectorSubcoreMesh(core_axis_name="c", subcore_axis_name="s")
    n = users.shape[0]
    b_per_w = n // _NW
    out_type = (
        jax.ShapeDtypeStruct((n, NF), jnp.float32),
        jax.ShapeDtypeStruct((n, NF), jnp.float32),
    )

    @pl.kernel(
        out_type=out_type,
        mesh=mesh,
        scratch_types=[
            pltpu.VMEM((b_per_w,), jnp.int32),
            pltpu.VMEM((b_per_w,), jnp.int32),
            pltpu.SemaphoreType.DMA,
        ],
    )
    def gather_kernel(u_idx_hbm, i_idx_hbm, u_tab_hbm, i_tab_hbm,
                      u_out_hbm, i_out_hbm, uidx_s, iidx_s, sem):
        wid = jax.lax.axis_index("s") * _NC + jax.lax.axis_index("c")
        base = wid * b_per_w
        pltpu.sync_copy(u_idx_hbm.at[pl.ds(base, b_per_w)], uidx_s)
        pltpu.sync_copy(i_idx_hbm.at[pl.ds(base, b_per_w)], iidx_s)

        @pl.loop(0, b_per_w, step=16)
        def _issue(k):
            vu = uidx_s[pl.ds(k, 16)]
            vi = iidx_s[pl.ds(k, 16)]
            for j in range(16):
                pltpu.async_copy(u_tab_hbm.at[pl.ds(vu[j], 1)],
                                 u_out_hbm.at[pl.ds(base + k + j, 1)], sem)
                pltpu.async_copy(i_tab_hbm.at[pl.ds(vi[j], 1)],
                                 i_out_hbm.at[pl.ds(base + k + j, 1)], sem)

        @pl.loop(0, b_per_w)
        def _drain(k):
            pltpu.make_async_copy(u_tab_hbm.at[pl.ds(0, 1)],
                                  u_out_hbm.at[pl.ds(base + k, 1)],
                                  sem).wait()
            pltpu.make_async_copy(i_tab_hbm.at[pl.ds(0, 1)],
                                  i_out_hbm.at[pl.ds(base + k, 1)],
                                  sem).wait()

    return gather_kernel(users, items, user_table, item_table)


_BT = 2048


def _mlp_body(u_ref, i_ref, w1u_ref, w1i_ref, b1_ref, w2_ref, b2_ref,
              w3_ref, b3_ref, out_ref):
    h = jnp.dot(u_ref[...], w1u_ref[...], preferred_element_type=jnp.float32)
    h += jnp.dot(i_ref[...], w1i_ref[...], preferred_element_type=jnp.float32)
    h = jnp.maximum(h + b1_ref[...], 0.0)
    h = jnp.dot(h, w2_ref[...], preferred_element_type=jnp.float32)
    h = jnp.maximum(h + b2_ref[...], 0.0)
    out_ref[...] = (
        jnp.dot(h, w3_ref[...], preferred_element_type=jnp.float32)
        + b3_ref[...]
    )


def _tc_mlp(u, i, W1, b1, W2, b2, W3, b3):
    n = u.shape[0]
    w1u = W1[:NF]
    w1i = W1[NF:]
    grid = (n // _BT,)
    full = lambda *shape: pl.BlockSpec(shape, lambda g: (0,) * len(shape))
    out = pl.pallas_call(
        _mlp_body,
        grid=grid,
        in_specs=[
            pl.BlockSpec((_BT, NF), lambda g: (g, 0)),
            pl.BlockSpec((_BT, NF), lambda g: (g, 0)),
            full(NF, W1.shape[1]),
            full(NF, W1.shape[1]),
            full(1, b1.shape[0]),
            full(W2.shape[0], W2.shape[1]),
            full(1, b2.shape[0]),
            full(W3.shape[0], W3.shape[1]),
            full(1, 1),
        ],
        out_specs=pl.BlockSpec((_BT, 1), lambda g: (g, 0)),
        out_shape=jax.ShapeDtypeStruct((n, 1), jnp.float32),
    )(u, i, w1u, w1i, b1.reshape(1, -1), W2, b2.reshape(1, -1), W3,
      b3.reshape(1, 1))
    return out.reshape(n)


def kernel(users, items, user_table, item_table, W1, b1, W2, b2, W3, b3):
    users = users.astype(jnp.int32)
    items = items.astype(jnp.int32)
    u, i = _sc_gather_pair(users, items, user_table, item_table)
    return _tc_mlp(u, i, W1, b1, W2, b2, W3, b3)


# speedup vs baseline: 1.6769x; 1.6769x over previous
import jax
import jax.numpy as jnp
from jax.experimental import pallas as pl
from jax.experimental.pallas import tpu as pltpu
from jax.experimental.pallas import tpu_sc as plsc

BATCH = 16384
NF = 64


_NC = 2
_NS = 16
_NW = _NC * _NS


def _sc_gather_pair(users, items, user_table, item_table):
    mesh = plsc.VectorSubcoreMesh(core_axis_name="c", subcore_axis_name="s")
    n = users.shape[0]
    b_per_w = n // _NW
    out_type = (
        jax.ShapeDtypeStruct((n, NF), jnp.float32),
        jax.ShapeDtypeStruct((n, NF), jnp.float32),
    )

    @pl.kernel(
        out_type=out_type,
        mesh=mesh,
        scratch_types=[
            pltpu.VMEM((b_per_w,), jnp.int32),
            pltpu.VMEM((b_per_w,), jnp.int32),
            pltpu.VMEM((b_per_w, NF), jnp.float32),
            pltpu.SemaphoreType.DMA,
        ],
    )
    def gather_kernel(u_idx_hbm, i_idx_hbm, u_tab_hbm, i_tab_hbm,
                      u_out_hbm, i_out_hbm, uidx_s, iidx_s, rows_v, sem):
        wid = jax.lax.axis_index("s") * _NC + jax.lax.axis_index("c")
        base = wid * b_per_w
        pltpu.sync_copy(u_idx_hbm.at[pl.ds(base, b_per_w)], uidx_s)
        pltpu.sync_copy(i_idx_hbm.at[pl.ds(base, b_per_w)], iidx_s)

        def one_table(idx_ref, tab_hbm, out_hbm):
            @pl.loop(0, b_per_w, step=16)
            def _issue(k):
                v = idx_ref[pl.ds(k, 16)]
                for j in range(16):
                    pltpu.async_copy(tab_hbm.at[pl.ds(v[j], 1)],
                                     rows_v.at[pl.ds(k + j, 1)], sem)

            @pl.loop(0, b_per_w)
            def _drain(k):
                pltpu.make_async_copy(tab_hbm.at[pl.ds(0, 1)],
                                      rows_v.at[pl.ds(k, 1)], sem).wait()

            pltpu.sync_copy(rows_v, out_hbm.at[pl.ds(base, b_per_w)])

        one_table(uidx_s, u_tab_hbm, u_out_hbm)
        one_table(iidx_s, i_tab_hbm, i_out_hbm)

    return gather_kernel(users, items, user_table, item_table)


_BT = 2048


def _mlp_body(u_ref, i_ref, w1u_ref, w1i_ref, b1_ref, w2_ref, b2_ref,
              w3_ref, b3_ref, out_ref):
    h = jnp.dot(u_ref[...], w1u_ref[...], preferred_element_type=jnp.float32)
    h += jnp.dot(i_ref[...], w1i_ref[...], preferred_element_type=jnp.float32)
    h = jnp.maximum(h + b1_ref[...], 0.0)
    h = jnp.dot(h, w2_ref[...], preferred_element_type=jnp.float32)
    h = jnp.maximum(h + b2_ref[...], 0.0)
    out_ref[...] = (
        jnp.dot(h, w3_ref[...], preferred_element_type=jnp.float32)
        + b3_ref[...]
    )


def _tc_mlp(u, i, W1, b1, W2, b2, W3, b3):
    n = u.shape[0]
    w1u = W1[:NF]
    w1i = W1[NF:]
    grid = (n // _BT,)
    full = lambda *shape: pl.BlockSpec(shape, lambda g: (0,) * len(shape))
    out = pl.pallas_call(
        _mlp_body,
        grid=grid,
        in_specs=[
            pl.BlockSpec((_BT, NF), lambda g: (g, 0)),
            pl.BlockSpec((_BT, NF), lambda g: (g, 0)),
            full(NF, W1.shape[1]),
            full(NF, W1.shape[1]),
            full(1, b1.shape[0]),
            full(W2.shape[0], W2.shape[1]),
            full(1, b2.shape[0]),
            full(W3.shape[0], W3.shape[1]),
            full(1, 1),
        ],
        out_specs=pl.BlockSpec((_BT, 1), lambda g: (g, 0)),
        out_shape=jax.ShapeDtypeStruct((n, 1), jnp.float32),
    )(u, i, w1u, w1i, b1.reshape(1, -1), W2, b2.reshape(1, -1), W3,
      b3.reshape(1, 1))
    return out.reshape(n)


def kernel(users, items, user_table, item_table, W1, b1, W2, b2, W3, b3):
    users = users.astype(jnp.int32)
    items = items.astype(jnp.int32)
    u, i = _sc_gather_pair(users, items, user_table, item_table)
    return _tc_mlp(u, i, W1, b1, W2, b2, W3, b3)


# speedup vs baseline: 1.6795x; 1.0016x over previous
import jax
import jax.numpy as jnp
from jax.experimental import pallas as pl
from jax.experimental.pallas import tpu as pltpu
from jax.experimental.pallas import tpu_sc as plsc

BATCH = 16384
NF = 64


_NC = 2
_NS = 16
_NW = _NC * _NS


def _sc_gather_pair(users, items, user_table, item_table):
    mesh = plsc.VectorSubcoreMesh(core_axis_name="c", subcore_axis_name="s")
    n = users.shape[0]
    b_per_w = n // _NW
    out_type = (
        jax.ShapeDtypeStruct((n, NF), jnp.float32),
        jax.ShapeDtypeStruct((n, NF), jnp.float32),
    )

    @pl.kernel(
        out_type=out_type,
        mesh=mesh,
        scratch_types=[
            pltpu.VMEM((b_per_w,), jnp.int32),
            pltpu.VMEM((b_per_w,), jnp.int32),
            pltpu.VMEM((b_per_w, NF), jnp.float32),
            pltpu.SemaphoreType.DMA,
        ],
    )
    def gather_kernel(u_idx_hbm, i_idx_hbm, u_tab_hbm, i_tab_hbm,
                      u_out_hbm, i_out_hbm, uidx_s, iidx_s, rows_v, sem):
        wid = jax.lax.axis_index("s") * _NC + jax.lax.axis_index("c")
        base = wid * b_per_w
        pltpu.sync_copy(u_idx_hbm.at[pl.ds(base, b_per_w)], uidx_s)
        pltpu.sync_copy(i_idx_hbm.at[pl.ds(base, b_per_w)], iidx_s)

        def one_table(idx_ref, tab_hbm, out_hbm):
            @pl.loop(0, b_per_w, step=16)
            def _issue(k):
                v = idx_ref[pl.ds(k, 16)]
                for j in range(16):
                    pltpu.async_copy(tab_hbm.at[pl.ds(v[j], 1)],
                                     rows_v.at[pl.ds(k + j, 1)], sem)

            @pl.loop(0, b_per_w, step=128)
            def _drain(k):
                pltpu.make_async_copy(tab_hbm.at[pl.ds(0, 128)],
                                      rows_v.at[pl.ds(k, 128)], sem).wait()

            pltpu.sync_copy(rows_v, out_hbm.at[pl.ds(base, b_per_w)])

        one_table(uidx_s, u_tab_hbm, u_out_hbm)
        one_table(iidx_s, i_tab_hbm, i_out_hbm)

    return gather_kernel(users, items, user_table, item_table)


_BT = 2048


def _mlp_body(u_ref, i_ref, w1u_ref, w1i_ref, b1_ref, w2_ref, b2_ref,
              w3_ref, b3_ref, out_ref):
    h = jnp.dot(u_ref[...], w1u_ref[...], preferred_element_type=jnp.float32)
    h += jnp.dot(i_ref[...], w1i_ref[...], preferred_element_type=jnp.float32)
    h = jnp.maximum(h + b1_ref[...], 0.0)
    h = jnp.dot(h, w2_ref[...], preferred_element_type=jnp.float32)
    h = jnp.maximum(h + b2_ref[...], 0.0)
    out_ref[...] = (
        jnp.dot(h, w3_ref[...], preferred_element_type=jnp.float32)
        + b3_ref[...]
    )


def _tc_mlp(u, i, W1, b1, W2, b2, W3, b3):
    n = u.shape[0]
    w1u = W1[:NF]
    w1i = W1[NF:]
    grid = (n // _BT,)
    full = lambda *shape: pl.BlockSpec(shape, lambda g: (0,) * len(shape))
    out = pl.pallas_call(
        _mlp_body,
        grid=grid,
        in_specs=[
            pl.BlockSpec((_BT, NF), lambda g: (g, 0)),
            pl.BlockSpec((_BT, NF), lambda g: (g, 0)),
            full(NF, W1.shape[1]),
            full(NF, W1.shape[1]),
            full(1, b1.shape[0]),
            full(W2.shape[0], W2.shape[1]),
            full(1, b2.shape[0]),
            full(W3.shape[0], W3.shape[1]),
            full(1, 1),
        ],
        out_specs=pl.BlockSpec((_BT, 1), lambda g: (g, 0)),
        out_shape=jax.ShapeDtypeStruct((n, 1), jnp.float32),
    )(u, i, w1u, w1i, b1.reshape(1, -1), W2, b2.reshape(1, -1), W3,
      b3.reshape(1, 1))
    return out.reshape(n)


def kernel(users, items, user_table, item_table, W1, b1, W2, b2, W3, b3):
    users = users.astype(jnp.int32)
    items = items.astype(jnp.int32)
    u, i = _sc_gather_pair(users, items, user_table, item_table)
    return _tc_mlp(u, i, W1, b1, W2, b2, W3, b3)
